# baseline (speedup 1.0000x reference)
import functools

import jax
import jax.numpy as jnp
from jax import lax
from jax.experimental import pallas as pl
from jax.experimental.pallas import tpu as pltpu
from jax.experimental.pallas import tpu_sc as plsc

D = 256
DH = 128
NC = 2
NS = 16
LANES = 16
CHUNK = 64
NBUF = 5
TN = 1000


def _mm_body(x_ref, w_ref, o_ref):
    o_ref[...] = jnp.dot(x_ref[...], w_ref[...],
                         preferred_element_type=jnp.float32)


def _matmul_split(x, w):
    n = x.shape[0]
    nt = n // TN
    return pl.pallas_call(
        _mm_body,
        grid=(NC, nt),
        in_specs=[
            pl.BlockSpec((TN, D), lambda c, t: (t, 0)),
            pl.BlockSpec((D, DH), lambda c, t: (0, c)),
        ],
        out_specs=pl.BlockSpec((TN, DH), lambda c, t: (c * nt + t, 0)),
        out_shape=jax.ShapeDtypeStruct((NC * n, DH), jnp.float32),
    )(x, w)


def _spmm_sc(sup_cat, epk, b2, n):
    nchunks = epk.shape[0] // NS
    rows_per_sub = 624
    rem_rows = n - NS * rows_per_sub
    binit_rows = rows_per_sub // 13

    mesh = plsc.VectorSubcoreMesh(core_axis_name="c", subcore_axis_name="s",
                                  num_cores=NC, num_subcores=NS)

    @functools.partial(
        pl.kernel,
        out_type=jax.ShapeDtypeStruct((NC * n, DH), jnp.float32),
        mesh=mesh,
        scratch_types=[
            pltpu.VMEM_SHARED((n, DH), jnp.float32),
            [pltpu.VMEM((3, CHUNK), jnp.int32) for _ in range(NBUF)],
            [pltpu.VMEM((CHUNK, DH), jnp.float32) for _ in range(NBUF)],
            pltpu.VMEM((binit_rows, DH), jnp.float32),
            pltpu.VMEM((1, DH), jnp.float32),
            [pltpu.SemaphoreType.DMA for _ in range(NBUF)],
            [pltpu.SemaphoreType.DMA for _ in range(NBUF)],
            [pltpu.SemaphoreType.DMA for _ in range(NBUF)],
        ],
    )
    def k(sup_hbm, epk_hbm, b_hbm, out_hbm,
          acc, ebs, rws, brows, bvec, isems, gsems, ssems):
        c = lax.axis_index("c")
        s = lax.axis_index("s")
        off = c * n
        first = s * nchunks

        pltpu.sync_copy(b_hbm.at[c], bvec)
        bvs = [bvec[0, pl.ds(j * LANES, LANES)] for j in range(DH // LANES)]

        def bfill_body(r, carry):
            for j in range(DH // LANES):
                brows[r, pl.ds(j * LANES, LANES)] = bvs[j]
            return carry

        lax.fori_loop(0, binit_rows, bfill_body, 0)
        r0 = s * rows_per_sub
        for i in range(rows_per_sub // binit_rows):
            pltpu.sync_copy(brows, acc.at[pl.ds(r0 + i * binit_rows,
                                                binit_rows)])

        @pl.when(s == NS - 1)
        def _():
            pltpu.sync_copy(brows.at[pl.ds(0, rem_rows)],
                            acc.at[pl.ds(NS * rows_per_sub, rem_rows)])

        plsc.subcore_barrier()

        def idx_start(j, eb, sem):
            pltpu.async_copy(epk_hbm.at[first + j], eb, sem)

        def idx_wait(j, eb, sem):
            pltpu.make_async_copy(epk_hbm.at[first + j], eb, sem).wait()
            for g in range(CHUNK // LANES):
                sl = pl.ds(g * LANES, LANES)
                eb[0, sl] = eb[0, sl] + off

        def gather_start(eb, rw, sem):
            pltpu.async_copy(sup_hbm.at[eb.at[0]], rw, sem)

        def gather_wait(eb, rw, sem):
            pltpu.make_async_copy(sup_hbm.at[eb.at[0]], rw, sem).wait()

        def scatter_start(eb, rw, sem):
            pltpu.async_copy(rw, acc.at[eb.at[1]], sem, add=True)

        def scatter_wait(eb, rw, sem):
            pltpu.make_async_copy(rw, acc.at[eb.at[1]], sem).wait()

        def scale(eb, rw):
            def gbody(g, carry):
                wv = lax.bitcast_convert_type(eb[2, pl.ds(g * LANES, LANES)],
                                              jnp.float32)
                for kk in range(LANES):
                    wk = jnp.broadcast_to(wv[kk], (LANES,))
                    e = g * LANES + kk
                    for jj in range(DH // LANES):
                        sl = pl.ds(jj * LANES, LANES)
                        rw[e, sl] = rw[e, sl] * wk
                return carry

            lax.fori_loop(0, CHUNK // LANES, gbody, 0)

        bufs = [(ebs[p], rws[p], isems[p], gsems[p], ssems[p])
                for p in range(NBUF)]

        def B(q):
            eb, rw, isem, gsem, ssem = bufs[q % NBUF]
            return eb, rw, isem, gsem, ssem

        for q in range(3):
            eb, rw, isem, gsem, ssem = B(q)
            idx_start(q, eb, isem)
        for q in range(2):
            eb, rw, isem, gsem, ssem = B(q)
            idx_wait(q, eb, isem)
            gather_start(eb, rw, gsem)

        def ring_body(t, carry):
            for p in range(NBUF):
                j = t * NBUF + p
                ebw, rww, _, _, ssw = B(p + 3)

                @pl.when(j >= 2)
                def _():
                    scatter_wait(ebw, rww, ssw)

                @pl.when(j + 3 < nchunks)
                def _():
                    idx_start(j + 3, ebw, B(p + 3)[2])

                ebg, rwg, isg, gsg, _ = B(p + 2)

                @pl.when(j + 2 < nchunks)
                def _():
                    idx_wait(j + 2, ebg, isg)
                    gather_start(ebg, rwg, gsg)

                ebp, rwp, _, gsp, ssp = B(p)
                gather_wait(ebp, rwp, gsp)
                scale(ebp, rwp)
                scatter_start(ebp, rwp, ssp)
            return carry

        lax.fori_loop(0, nchunks // NBUF, ring_body, 0)
        for q in (nchunks - 2, nchunks - 1):
            eb, rw, _, _, ssem = B(q)
            scatter_wait(eb, rw, ssem)
        plsc.subcore_barrier()

        pltpu.sync_copy(acc.at[pl.ds(r0, rows_per_sub)],
                        out_hbm.at[pl.ds(c * n + r0, rows_per_sub)])

        @pl.when(s == NS - 1)
        def _():
            pltpu.sync_copy(acc.at[pl.ds(NS * rows_per_sub, rem_rows)],
                            out_hbm.at[pl.ds(c * n + NS * rows_per_sub,
                                             rem_rows)])

    return k(sup_cat, epk, b2)


@jax.jit
def kernel(x, edge_index, edge_weight, W, b):
    n = x.shape[0]
    e = edge_weight.shape[0]
    sup = _matmul_split(x, W)

    dst = edge_index[0].astype(jnp.int32)
    src = edge_index[1].astype(jnp.int32)
    w = edge_weight.astype(jnp.float32)

    step = NS * CHUNK * NBUF
    epad = -(-e // step) * step
    pad = epad - e
    if pad:
        src = jnp.pad(src, (0, pad))
        dst = jnp.pad(dst, (0, pad))
        w = jnp.pad(w, (0, pad))
    epk = jnp.stack([src, dst, w.view(jnp.int32)], axis=0)
    epk = epk.reshape(3, epad // CHUNK, CHUNK).transpose(1, 0, 2)

    out_cat = _spmm_sc(sup, epk, b.reshape(NC, 1, DH), n)
    return jnp.concatenate([out_cat[:n], out_cat[n:]], axis=1)


# speedup vs baseline: 1.1258x; 1.1258x over previous
import functools

import jax
import jax.numpy as jnp
from jax import lax
from jax.experimental import pallas as pl
from jax.experimental.pallas import tpu as pltpu
from jax.experimental.pallas import tpu_sc as plsc

D = 256
DH = 128
NC = 2
NS = 16
LANES = 16
CHUNK = 64
NBUF = 5
TN = 1000


def _mm_body(x_ref, w_ref, o_ref):
    o_ref[...] = jnp.dot(x_ref[...], w_ref[...],
                         preferred_element_type=jnp.float32)


def _matmul_split(x, w):
    n = x.shape[0]
    nt = n // TN
    return pl.pallas_call(
        _mm_body,
        grid=(NC, nt),
        in_specs=[
            pl.BlockSpec((TN, D), lambda c, t: (t, 0)),
            pl.BlockSpec((D, DH), lambda c, t: (0, c)),
        ],
        out_specs=pl.BlockSpec((TN, DH), lambda c, t: (c * nt + t, 0)),
        out_shape=jax.ShapeDtypeStruct((NC * n, DH), jnp.float32),
    )(x, w)


def _spmm_sc(sup_cat, epk, b2, n):
    nchunks = epk.shape[0] // NS
    rows_per_sub = 624
    rem_rows = n - NS * rows_per_sub
    binit_rows = 48

    mesh = plsc.VectorSubcoreMesh(core_axis_name="c", subcore_axis_name="s",
                                  num_cores=NC, num_subcores=NS)

    @functools.partial(
        pl.kernel,
        out_type=jax.ShapeDtypeStruct((n, D), jnp.float32),
        mesh=mesh,
        scratch_types=[
            pltpu.VMEM_SHARED((n, DH), jnp.float32),
            [pltpu.VMEM((3, CHUNK), jnp.int32) for _ in range(NBUF)],
            [pltpu.VMEM((CHUNK, DH), jnp.float32) for _ in range(NBUF)],
            pltpu.VMEM((binit_rows, DH), jnp.float32),
            pltpu.VMEM((1, DH), jnp.float32),
            [pltpu.SemaphoreType.DMA for _ in range(NBUF)],
            [pltpu.SemaphoreType.DMA for _ in range(NBUF)],
            [pltpu.SemaphoreType.DMA for _ in range(NBUF)],
        ],
    )
    def k(sup_hbm, epk_hbm, b_hbm, out_hbm,
          acc, ebs, rws, brows, bvec, isems, gsems, ssems):
        c = lax.axis_index("c")
        s = lax.axis_index("s")
        off = c * n
        first = s * nchunks

        pltpu.sync_copy(b_hbm.at[c], bvec)
        bvs = [bvec[0, pl.ds(j * LANES, LANES)] for j in range(DH // LANES)]

        def bfill_body(r, carry):
            for j in range(DH // LANES):
                brows[r, pl.ds(j * LANES, LANES)] = bvs[j]
            return carry

        lax.fori_loop(0, binit_rows, bfill_body, 0)
        r0 = s * rows_per_sub
        for i in range(rows_per_sub // binit_rows):
            pltpu.sync_copy(brows, acc.at[pl.ds(r0 + i * binit_rows,
                                                binit_rows)])

        @pl.when(s == NS - 1)
        def _():
            pltpu.sync_copy(brows.at[pl.ds(0, rem_rows)],
                            acc.at[pl.ds(NS * rows_per_sub, rem_rows)])

        plsc.subcore_barrier()

        def idx_start(j, eb, sem):
            pltpu.async_copy(epk_hbm.at[first + j], eb, sem)

        def idx_wait(j, eb, sem):
            pltpu.make_async_copy(epk_hbm.at[first + j], eb, sem).wait()
            for g in range(CHUNK // LANES):
                sl = pl.ds(g * LANES, LANES)
                eb[0, sl] = eb[0, sl] + off

        def gather_start(eb, rw, sem):
            pltpu.async_copy(sup_hbm.at[eb.at[0]], rw, sem)

        def gather_wait(eb, rw, sem):
            pltpu.make_async_copy(sup_hbm.at[eb.at[0]], rw, sem).wait()

        def scatter_start(eb, rw, sem):
            pltpu.async_copy(rw, acc.at[eb.at[1]], sem, add=True)

        def scatter_wait(eb, rw, sem):
            pltpu.make_async_copy(rw, acc.at[eb.at[1]], sem).wait()

        def scale(eb, rw):
            def gbody(g, carry):
                wv = lax.bitcast_convert_type(eb[2, pl.ds(g * LANES, LANES)],
                                              jnp.float32)
                for kk in range(LANES):
                    wk = jnp.broadcast_to(wv[kk], (LANES,))
                    e = g * LANES + kk
                    for jj in range(DH // LANES):
                        sl = pl.ds(jj * LANES, LANES)
                        rw[e, sl] = rw[e, sl] * wk
                return carry

            lax.fori_loop(0, CHUNK // LANES, gbody, 0)

        def B(q):
            p = q % NBUF
            return ebs[p], rws[p], isems[p], gsems[p], ssems[p]

        for q in range(3):
            eb, rw, isem, gsem, ssem = B(q)
            idx_start(q, eb, isem)
        for q in range(2):
            eb, rw, isem, gsem, ssem = B(q)
            idx_wait(q, eb, isem)
            gather_start(eb, rw, gsem)

        def ring_body(t, carry):
            for p in range(NBUF):
                j = t * NBUF + p
                ebw, rww, isw, _, ssw = B(p + 3)

                @pl.when(j >= 2)
                def _():
                    scatter_wait(ebw, rww, ssw)

                @pl.when(j + 3 < nchunks)
                def _():
                    idx_start(j + 3, ebw, isw)

                ebg, rwg, isg, gsg, _ = B(p + 2)

                @pl.when(j + 2 < nchunks)
                def _():
                    idx_wait(j + 2, ebg, isg)
                    gather_start(ebg, rwg, gsg)

                ebp, rwp, _, gsp, ssp = B(p)
                gather_wait(ebp, rwp, gsp)
                scale(ebp, rwp)
                scatter_start(ebp, rwp, ssp)
            return carry

        lax.fori_loop(0, nchunks // NBUF, ring_body, 0)
        for q in (nchunks - 2, nchunks - 1):
            eb, rw, _, _, ssem = B(q)
            scatter_wait(eb, rw, ssem)
        plsc.subcore_barrier()

        pltpu.sync_copy(acc.at[pl.ds(r0, rows_per_sub)],
                        out_hbm.at[pl.ds(r0, rows_per_sub),
                                   pl.ds(c * DH, DH)])

        @pl.when(s == NS - 1)
        def _():
            pltpu.sync_copy(acc.at[pl.ds(NS * rows_per_sub, rem_rows)],
                            out_hbm.at[pl.ds(NS * rows_per_sub, rem_rows),
                                       pl.ds(c * DH, DH)])

    return k(sup_cat, epk, b2)


@jax.jit
def kernel(x, edge_index, edge_weight, W, b):
    n = x.shape[0]
    e = edge_weight.shape[0]
    sup = _matmul_split(x, W)

    dst = edge_index[0].astype(jnp.int32)
    src = edge_index[1].astype(jnp.int32)
    w = edge_weight.astype(jnp.float32)

    step = NS * CHUNK * NBUF
    epad = -(-e // step) * step
    pad = epad - e
    if pad:
        src = jnp.pad(src, (0, pad))
        dst = jnp.pad(dst, (0, pad))
        w = jnp.pad(w, (0, pad))
    epk = jnp.stack([src, dst, w.view(jnp.int32)], axis=0)
    epk = epk.reshape(3, epad // CHUNK, CHUNK).transpose(1, 0, 2)

    return _spmm_sc(sup, epk, b.reshape(NC, 1, DH), n)
